# Initial kernel scaffold; baseline (speedup 1.0000x reference)
import jax
import jax.numpy as jnp
from jax.experimental import pallas as pl


def kernel(x_input, ln_gamma, ln_beta):
    raise NotImplementedError("write your pallas kernel here")


# speedup vs baseline: 38.4990x; 38.4990x over previous
import functools

import jax
import jax.numpy as jnp
from jax.experimental import pallas as pl

_SPARSITY = 0.15
_GAIN = 3.0
_EPS = 1e-5


def _tc_kernel(x_ref, g_ref, b_ref, o_ref, *, k):
    x = x_ref[...]
    n = x.shape[1]

    mean = jnp.mean(x, axis=1, keepdims=True)
    xc = x - mean
    var = jnp.mean(xc * xc, axis=1, keepdims=True)
    xn = xc * jax.lax.rsqrt(var + _EPS)
    xn = xn * g_ref[...] + b_ref[...]

    u = jax.lax.bitcast_convert_type(xn, jnp.uint32)
    neg = u >= jnp.uint32(0x80000000)
    s = jnp.where(neg, ~u, u | jnp.uint32(0x80000000))

    rows = x.shape[0]
    lo0 = jnp.zeros((rows, 1), jnp.uint32)
    hi0 = jnp.full((rows, 1), 0xFFFFFFFF, jnp.uint32)

    def body(_, carry):
        lo, hi = carry
        d = hi - lo
        mid = lo + (d >> 1) + (d & jnp.uint32(1))
        cnt = jnp.sum((s >= mid).astype(jnp.int32), axis=1, keepdims=True)
        pred = cnt >= k
        lo = jnp.where(pred, mid, lo)
        hi = jnp.where(pred, hi, mid - jnp.uint32(1))
        return lo, hi

    lo, _ = jax.lax.fori_loop(0, 32, body, (lo0, hi0))
    keep = s >= lo

    m1 = jnp.max(xn, axis=1, keepdims=True)
    is_max = xn == m1
    nmax = jnp.sum(is_max.astype(jnp.int32), axis=1, keepdims=True)
    m2_strict = jnp.max(jnp.where(is_max, -jnp.inf, xn), axis=1, keepdims=True)
    m2 = jnp.where(nmax >= 2, m1, m2_strict)
    gain = jax.nn.sigmoid(m1 - m2) * _GAIN + 1.0

    o_ref[...] = jnp.where(keep, xn * gain, 0.0)


@jax.jit
def kernel(x_input, ln_gamma, ln_beta):
    b, n = x_input.shape
    k = max(int(n * _SPARSITY), 2)
    rb = 16
    grid = (b // rb,)
    body = functools.partial(_tc_kernel, k=k)
    return pl.pallas_call(
        body,
        grid=grid,
        in_specs=[
            pl.BlockSpec((rb, n), lambda i: (i, 0)),
            pl.BlockSpec((1, n), lambda i: (0, 0)),
            pl.BlockSpec((1, n), lambda i: (0, 0)),
        ],
        out_specs=pl.BlockSpec((rb, n), lambda i: (i, 0)),
        out_shape=jax.ShapeDtypeStruct((b, n), jnp.float32),
    )(x_input, ln_gamma.reshape(1, n), ln_beta.reshape(1, n))


# speedup vs baseline: 45.7366x; 1.1880x over previous
import functools

import jax
import jax.numpy as jnp
from jax.experimental import pallas as pl

_SPARSITY = 0.15
_GAIN = 3.0
_EPS = 1e-5


def _tc_kernel(x_ref, g_ref, b_ref, o_ref, *, k):
    x = x_ref[...]
    rows = x.shape[0]

    mean = jnp.mean(x, axis=1, keepdims=True)
    xc = x - mean
    var = jnp.mean(xc * xc, axis=1, keepdims=True)
    xn = xc * jax.lax.rsqrt(var + _EPS)
    xn = xn * g_ref[...] + b_ref[...]

    u = jax.lax.bitcast_convert_type(xn, jnp.uint32)
    neg = u >= jnp.uint32(0x80000000)
    s = jnp.where(neg, ~u, u | jnp.uint32(0x80000000))

    lo0 = jnp.zeros((rows, 1), jnp.uint32)
    hi0 = jnp.full((rows, 1), 0xFFFFFFFF, jnp.uint32)

    def cond(carry):
        i, lo, hi = carry
        return jnp.logical_and(i < 32, jnp.logical_not(jnp.all(lo == hi)))

    def body(carry):
        i, lo, hi = carry
        d = hi - lo
        mid = lo + (d >> 1) + (d & jnp.uint32(1))
        cnt = jnp.sum((s >= mid).astype(jnp.int32), axis=1, keepdims=True)
        pred = cnt >= k
        lo = jnp.where(pred, mid, lo)
        hi = jnp.where(cnt == k, mid, jnp.where(pred, hi, mid - jnp.uint32(1)))
        return i + 1, lo, hi

    _, lo, _ = jax.lax.while_loop(cond, body, (jnp.int32(0), lo0, hi0))
    keep = s >= lo

    m1 = jnp.max(xn, axis=1, keepdims=True)
    is_max = xn == m1
    nmax = jnp.sum(is_max.astype(jnp.int32), axis=1, keepdims=True)
    m2_strict = jnp.max(jnp.where(is_max, -jnp.inf, xn), axis=1, keepdims=True)
    m2 = jnp.where(nmax >= 2, m1, m2_strict)
    gain = jax.nn.sigmoid(m1 - m2) * _GAIN + 1.0

    o_ref[...] = jnp.where(keep, xn * gain, 0.0)


@jax.jit
def kernel(x_input, ln_gamma, ln_beta):
    b, n = x_input.shape
    k = max(int(n * _SPARSITY), 2)
    rb = 32
    grid = (b // rb,)
    body = functools.partial(_tc_kernel, k=k)
    return pl.pallas_call(
        body,
        grid=grid,
        in_specs=[
            pl.BlockSpec((rb, n), lambda i: (i, 0)),
            pl.BlockSpec((1, n), lambda i: (0, 0)),
            pl.BlockSpec((1, n), lambda i: (0, 0)),
        ],
        out_specs=pl.BlockSpec((rb, n), lambda i: (i, 0)),
        out_shape=jax.ShapeDtypeStruct((b, n), jnp.float32),
    )(x_input, ln_gamma.reshape(1, n), ln_beta.reshape(1, n))
